# Initial kernel scaffold; baseline (speedup 1.0000x reference)
import jax
import jax.numpy as jnp
from jax.experimental import pallas as pl


def kernel(pred_hmp, pred_reg, pad_size, img_size):
    raise NotImplementedError("write your pallas kernel here")


# speedup vs baseline: 1.8867x; 1.8867x over previous
import functools

import jax
import jax.numpy as jnp
from jax import lax
from jax.experimental import pallas as pl
from jax.experimental.pallas import tpu as pltpu
from jax.experimental.pallas import tpu_sc as plsc

DOWN_RATIO = 4
K = 100
DET_THRESHOLD = 0.2

B, C, H, W = 8, 3, 96, 320
HW = H * W
N = C * HW
NC, NS = 2, 16
TPB = 4
CHUNK = N // TPB
NV = CHUNK // 16
CAP = 512
NKV = 112
MCAP = TPB * CAP
IDX_SENTINEL = 0x7FFFFFFF


def _nms_key_body(hm_ref, key_ref):
    x = hm_ref[0]
    neg = jnp.full((1, W), -jnp.inf, jnp.float32)
    up = jnp.concatenate([x[1:, :], neg], axis=0)
    dn = jnp.concatenate([neg, x[:-1, :]], axis=0)
    v = jnp.maximum(jnp.maximum(x, up), dn)
    negc = jnp.full((H, 1), -jnp.inf, jnp.float32)
    lf = jnp.concatenate([v[:, 1:], negc], axis=1)
    rt = jnp.concatenate([negc, v[:, :-1]], axis=1)
    m = jnp.maximum(jnp.maximum(v, lf), rt)
    score = jnp.where(m == x, x, jnp.float32(0.0))
    bits = lax.bitcast_convert_type(score, jnp.int32)
    key = jnp.where(bits < 0, ~bits, bits | jnp.int32(-(2 ** 31)))
    key_ref[0] = key


def _nms_keys(hm):
    return pl.pallas_call(
        _nms_key_body,
        grid=(B * C,),
        in_specs=[pl.BlockSpec((1, H, W), lambda i: (i, 0, 0))],
        out_specs=pl.BlockSpec((1, H, W), lambda i: (i, 0, 0)),
        out_shape=jax.ShapeDtypeStruct((B * C, H, W), jnp.int32),
    )(hm)


def _select_body(keys_hbm, reg_hbm, pad_hbm, img_hbm, out_hbm,
                 keys_v, hist_v, redh_v, tmp_v, ghv_v, ck_v, ci_v, cnt16_v,
                 tk_v, ti_v, mk_v, mi_v, rank_v, sel_v, skey_v, sidx_v,
                 gidx_v, greg_v, pad_v, img_v, out_v,
                 hist_s, candk_s, candi_s, cnt_s, sem):
    cid = lax.axis_index("c")
    sid = lax.axis_index("s")
    b = cid * (B // NC) + sid // TPB
    sub = sid % TPB
    s0 = (sid // TPB) * TPB
    base = sub * CHUNK

    lane = lax.iota(jnp.int32, 16)
    zeros16 = jnp.zeros((16,), jnp.int32)
    ones16 = jnp.ones((16,), jnp.int32)

    pltpu.sync_copy(keys_hbm.at[b, pl.ds(base, CHUNK)], keys_v)
    pltpu.sync_copy(pad_hbm, pad_v)
    pltpu.sync_copy(img_hbm, img_v)

    prefix = jnp.uint32(0)
    kr = jnp.int32(K)
    for p in range(3):
        shift = 24 - 8 * p

        def zero_hist(i, _):
            hist_v[pl.ds(i * 16, 16)] = zeros16
            return 0
        lax.fori_loop(0, 256, zero_hist, 0)

        pref = prefix

        def scan(i, _):
            ku = plsc.bitcast(keys_v[pl.ds(i * 16, 16)], jnp.uint32)
            d = ((ku >> jnp.uint32(shift)) & jnp.uint32(0xFF)).astype(jnp.int32)
            idx = lane * 256 + d
            if p == 0:
                plsc.addupdate_scatter(hist_v, [idx], ones16)
            else:
                msk = (ku >> jnp.uint32(shift + 8)) == pref
                plsc.addupdate_scatter(hist_v, [idx], ones16, mask=msk)
            return 0
        lax.fori_loop(0, NV, scan, 0)

        def reduce_lanes(dv, _):
            acc = zeros16
            for l in range(16):
                acc = acc + hist_v[pl.ds(l * 256 + dv * 16, 16)]
            redh_v[pl.ds(dv * 16, 16)] = acc
            return 0
        lax.fori_loop(0, 16, reduce_lanes, 0)

        pltpu.sync_copy(redh_v, hist_s.at[p, sid])
        plsc.subcore_barrier()

        pltpu.sync_copy(hist_s.at[p, s0], ghv_v)
        for j in range(1, TPB):
            pltpu.sync_copy(hist_s.at[p, s0 + j], tmp_v)
            for v in range(16):
                sl = pl.ds(v * 16, 16)
                ghv_v[sl] = ghv_v[sl] + tmp_v[sl]

        carry = jnp.int32(0)
        T = jnp.int32(0)
        aboveT = jnp.int32(0)
        for dv in range(15, -1, -1):
            v = ghv_v[pl.ds(dv * 16, 16)]
            incl = lax.rev(plsc.cumsum(lax.rev(v, (0,))), (0,)) + carry
            excl = incl - v
            m = (excl < kr) & (incl >= kr)
            has = jnp.max(jnp.where(m, 1, 0))
            dsel = jnp.max(jnp.where(m, lane, -1))
            esel = jnp.max(jnp.where(m, excl, -1))
            T = jnp.where(has == 1, dv * 16 + dsel, T)
            aboveT = jnp.where(has == 1, esel, aboveT)
            carry = jnp.max(incl)

        kr = kr - aboveT
        prefix = (prefix << jnp.uint32(8)) | T.astype(jnp.uint32)

    def prefill(i, _):
        ck_v[pl.ds(i * 16, 16)] = zeros16
        ci_v[pl.ds(i * 16, 16)] = jnp.full((16,), IDX_SENTINEL, jnp.int32)
        return 0
    lax.fori_loop(0, CAP // 16, prefill, 0)

    def compact(i, ptr):
        k16 = keys_v[pl.ds(i * 16, 16)]
        ku = plsc.bitcast(k16, jnp.uint32)
        m = (ku >> jnp.uint32(8)) >= prefix
        cs = plsc.cumsum(m.astype(jnp.int32))
        pos = ptr + cs - 1
        m2 = m & (pos < CAP)
        gi = base + i * 16 + lane
        plsc.store_scatter(ck_v, [pos], k16, mask=m2)
        plsc.store_scatter(ci_v, [pos], gi, mask=m2)
        return ptr + plsc.all_reduce_population_count(m)
    ptr = lax.fori_loop(0, NV, compact, zeros16)

    cnt16_v[...] = jnp.minimum(ptr, CAP)
    pltpu.sync_copy(ck_v, candk_s.at[sid])
    pltpu.sync_copy(ci_v, candi_s.at[sid])
    pltpu.sync_copy(cnt16_v, cnt_s.at[sid])
    plsc.subcore_barrier()

    @pl.when(sub == 0)
    def _leader():
        def prefill_m(i, _):
            sl = pl.ds(i * 16, 16)
            mk_v[sl] = zeros16
            mi_v[sl] = jnp.full((16,), IDX_SENTINEL, jnp.int32)
            rank_v[sl] = jnp.full((16,), jnp.int32(MCAP), jnp.int32)
            return 0
        lax.fori_loop(0, MCAP // 16 + 1, prefill_m, 0)

        off = jnp.int32(0)
        for j in range(TPB):
            pltpu.sync_copy(candk_s.at[s0 + j], tk_v)
            pltpu.sync_copy(candi_s.at[s0 + j], ti_v)
            pltpu.sync_copy(cnt_s.at[s0 + j], cnt16_v)
            cj = jnp.max(cnt16_v[...])

            def move(v, o):
                kk = tk_v[pl.ds(v * 16, 16)]
                ii = ti_v[pl.ds(v * 16, 16)]
                valid = (v * 16 + lane) < cj
                pos = o + v * 16 + lane
                plsc.store_scatter(mk_v, [pos], kk, mask=valid)
                plsc.store_scatter(mi_v, [pos], ii, mask=valid)
                return o
            lax.fori_loop(0, (cj + 15) // 16, move, off)
            off = off + cj

        ctot = off
        nvc = (ctot + 15) // 16

        def rank_one(i, _):
            kv = plsc.bitcast(jnp.full((16,), mk_v[pl.ds(i, 16)][0]),
                              jnp.uint32)
            iv = jnp.full((16,), mi_v[pl.ds(i, 16)][0])

            def inner(v, acc):
                kj = plsc.bitcast(mk_v[pl.ds(v * 16, 16)], jnp.uint32)
                ij = mi_v[pl.ds(v * 16, 16)]
                g = (kj > kv) | ((kj == kv) & (ij < iv))
                return acc + plsc.all_reduce_population_count(g)
            acc = lax.fori_loop(0, nvc, inner, zeros16)
            plsc.store_scatter(rank_v, [jnp.full((16,), i)], acc,
                               mask=lane == 0)
            return 0
        lax.fori_loop(0, ctot, rank_one, 0)

        for v in range(NKV // 16):
            sel_v[pl.ds(v * 16, 16)] = zeros16

        def build_sel(v, _):
            r16 = rank_v[pl.ds(v * 16, 16)]
            m = r16 < K
            plsc.store_scatter(sel_v, [r16], v * 16 + lane, mask=m)
            return 0
        lax.fori_loop(0, nvc, build_sel, 0)

        for v in range(NKV // 16):
            sl = pl.ds(v * 16, 16)
            s16 = sel_v[sl]
            skey_v[sl] = plsc.load_gather(mk_v, [s16])
            sidx_v[sl] = plsc.load_gather(mi_v, [s16])

        for v in range(NKV // 16):
            sl = pl.ds(v * 16, 16)
            idx = sidx_v[sl]
            cls = idx // HW
            pix = idx - cls * HW
            for c4 in range(4):
                gidx_v[c4, sl] = b * (4 * HW) + c4 * HW + pix
        copies = [
            pltpu.async_copy(reg_hbm.at[gidx_v.at[c4]], greg_v.at[c4], sem)
            for c4 in range(4)
        ]
        for cp in copies:
            cp.wait()

        pad16 = pad_v[...]
        img16 = img_v[...]
        padx = jnp.sum(jnp.where(lane == 2 * b, pad16, 0)).astype(jnp.float32)
        pady = jnp.sum(jnp.where(lane == 2 * b + 1, pad16, 0)).astype(jnp.float32)
        imgx = jnp.sum(jnp.where(lane == 2 * b, img16, 0)).astype(jnp.float32)
        imgy = jnp.sum(jnp.where(lane == 2 * b + 1, img16, 0)).astype(jnp.float32)

        for v in range(NKV // 16):
            sl = pl.ds(v * 16, 16)
            key = plsc.bitcast(skey_v[sl], jnp.uint32)
            idx = sidx_v[sl]
            pos_f = key >= jnp.uint32(0x80000000)
            sbits = jnp.where(pos_f, key & jnp.uint32(0x7FFFFFFF), ~key)
            score = plsc.bitcast(sbits, jnp.float32)
            cls = idx // HW
            pix = idx - cls * HW
            ysi = pix // W
            xsi = pix - ysi * W
            xs = xsi.astype(jnp.float32)
            ys = ysi.astype(jnp.float32)
            r0 = jnp.maximum(greg_v[0, sl], 0.0)
            r1 = jnp.maximum(greg_v[1, sl], 0.0)
            r2 = jnp.maximum(greg_v[2, sl], 0.0)
            r3 = jnp.maximum(greg_v[3, sl], 0.0)
            ratio = jnp.float32(DOWN_RATIO)
            x1 = (xs - r0) * ratio - padx
            y1 = (ys - r1) * ratio - pady
            x2 = (xs + r2) * ratio - padx
            y2 = (ys + r3) * ratio - pady
            x1 = jnp.clip(x1, 0.0, imgx - 1.0)
            y1 = jnp.clip(y1, 0.0, imgy - 1.0)
            x2 = jnp.clip(x2, 0.0, imgx - 1.0)
            y2 = jnp.clip(y2, 0.0, imgy - 1.0)
            msk = jnp.where(score >= jnp.float32(DET_THRESHOLD),
                            jnp.float32(1.0), jnp.float32(0.0))
            row = v * 16 + lane
            valid = row < K
            base6 = row * 6
            plsc.store_scatter(out_v, [base6], score * msk, mask=valid)
            plsc.store_scatter(out_v, [base6 + 1], x1 * msk, mask=valid)
            plsc.store_scatter(out_v, [base6 + 2], y1 * msk, mask=valid)
            plsc.store_scatter(out_v, [base6 + 3], x2 * msk, mask=valid)
            plsc.store_scatter(out_v, [base6 + 4], y2 * msk, mask=valid)
            plsc.store_scatter(out_v, [base6 + 5], cls.astype(jnp.float32),
                               mask=valid)

        pltpu.sync_copy(out_v, out_hbm.at[b])


_SELECT_SCRATCH = [
    pltpu.VMEM((CHUNK,), jnp.int32),
    pltpu.VMEM((4096,), jnp.int32),
    pltpu.VMEM((256,), jnp.int32),
    pltpu.VMEM((256,), jnp.int32),
    pltpu.VMEM((256,), jnp.int32),
    pltpu.VMEM((CAP,), jnp.int32),
    pltpu.VMEM((CAP,), jnp.int32),
    pltpu.VMEM((16,), jnp.int32),
    pltpu.VMEM((CAP,), jnp.int32),
    pltpu.VMEM((CAP,), jnp.int32),
    pltpu.VMEM((MCAP + 16,), jnp.int32),
    pltpu.VMEM((MCAP + 16,), jnp.int32),
    pltpu.VMEM((MCAP + 16,), jnp.int32),
    pltpu.VMEM((NKV,), jnp.int32),
    pltpu.VMEM((NKV,), jnp.int32),
    pltpu.VMEM((NKV,), jnp.int32),
    pltpu.VMEM((4, NKV), jnp.int32),
    pltpu.VMEM((4, NKV), jnp.float32),
    pltpu.VMEM((16,), jnp.int32),
    pltpu.VMEM((16,), jnp.int32),
    pltpu.VMEM((K * 6,), jnp.float32),
    pltpu.VMEM_SHARED((3, NS, 256), jnp.int32),
    pltpu.VMEM_SHARED((NS, CAP), jnp.int32),
    pltpu.VMEM_SHARED((NS, CAP), jnp.int32),
    pltpu.VMEM_SHARED((NS, 16), jnp.int32),
    pltpu.SemaphoreType.DMA,
]


def _select(keys, reg4, pad_flat, img_flat):
    mesh = plsc.VectorSubcoreMesh(core_axis_name="c", subcore_axis_name="s",
                                  num_cores=NC, num_subcores=NS)
    f = pl.kernel(
        _select_body,
        out_type=jax.ShapeDtypeStruct((B, K * 6), jnp.float32),
        mesh=mesh,
        scratch_types=_SELECT_SCRATCH,
        compiler_params=pltpu.CompilerParams(needs_layout_passes=False),
    )
    return f(keys, reg4, pad_flat, img_flat)


def kernel(pred_hmp, pred_reg, pad_size, img_size):
    keys = _nms_keys(pred_hmp.reshape(B * C, H, W)).reshape(B, N)
    reg4 = pred_reg[:, 0:4, :, :].reshape(B * 4 * HW)
    out = _select(keys, reg4, pad_size.reshape(2 * B), img_size.reshape(2 * B))
    return out.reshape(B, K, 6)
